# baseline (speedup 1.0000x reference)
import functools

import jax
import jax.numpy as jnp
from jax import lax
from jax.experimental import pallas as pl
from jax.experimental.pallas import tpu as pltpu
from jax.experimental.pallas import tpu_sc as plsc

_NC = 2
_NS = 16
_NW = _NC * _NS


def kernel(x):
    b, s, d = x.shape
    x2 = x.reshape(b * s, d)
    rows = b * s
    rows_per_w = rows // _NW
    mesh = plsc.VectorSubcoreMesh(core_axis_name="c", subcore_axis_name="s")

    @functools.partial(
        pl.kernel,
        out_type=jax.ShapeDtypeStruct((rows, d), x.dtype),
        mesh=mesh,
        scratch_types=[pltpu.SemaphoreType.DMA],
    )
    def sc_copy(x_hbm, o_hbm, sem):
        wid = lax.axis_index("s") * _NC + lax.axis_index("c")
        base = wid * rows_per_w
        pltpu.async_copy(
            x_hbm.at[pl.ds(base, rows_per_w)],
            o_hbm.at[pl.ds(base, rows_per_w)],
            sem,
        ).wait()

    return sc_copy(x2).reshape(b, s, d)


# speedup vs baseline: 35.9902x; 35.9902x over previous
import functools

import jax
import jax.numpy as jnp
from jax import lax
from jax.experimental import pallas as pl
from jax.experimental.pallas import tpu as pltpu
from jax.experimental.pallas import tpu_sc as plsc

_NC = 2
_NS = 16
_NW = _NC * _NS
_CHUNK = 32


def kernel(x):
    b, s, d = x.shape
    x2 = x.reshape(b * s, d)
    rows = b * s
    rows_per_w = rows // _NW
    nchunk = rows_per_w // _CHUNK
    mesh = plsc.VectorSubcoreMesh(core_axis_name="c", subcore_axis_name="s")

    @functools.partial(
        pl.kernel,
        out_type=jax.ShapeDtypeStruct((rows, d), x.dtype),
        mesh=mesh,
        scratch_types=[
            pltpu.VMEM((2, _CHUNK, d), x.dtype),
            pltpu.SemaphoreType.DMA((2,)),
            pltpu.SemaphoreType.DMA((2,)),
        ],
    )
    def sc_copy(x_hbm, o_hbm, bufs, lsems, ssems):
        wid = lax.axis_index("s") * _NC + lax.axis_index("c")
        base = wid * rows_per_w

        def load(c, bb):
            return pltpu.make_async_copy(
                x_hbm.at[pl.ds(base + c * _CHUNK, _CHUNK)], bufs.at[bb],
                lsems.at[bb])

        def store(c, bb):
            return pltpu.make_async_copy(
                bufs.at[bb], o_hbm.at[pl.ds(base + c * _CHUNK, _CHUNK)],
                ssems.at[bb])

        load(0, 0).start()
        for c in range(nchunk):
            bb = c % 2
            if c >= 1:
                store(c - 1, 1 - bb).wait()
            if c + 1 < nchunk:
                load(c + 1, 1 - bb).start()
            load(c, bb).wait()
            store(c, bb).start()
        store(nchunk - 1, (nchunk - 1) % 2).wait()

    return sc_copy(x2).reshape(b, s, d)


# speedup vs baseline: 49.3518x; 1.3713x over previous
import jax
import jax.numpy as jnp
from jax.experimental import pallas as pl
from jax.experimental.pallas import tpu as pltpu

_BLOCK_ROWS = 2048


def _copy_body(x_ref, o_ref):
    o_ref[...] = x_ref[...]


def kernel(x):
    b, s, d = x.shape
    x2 = x.reshape(b * s, d)
    grid = ((b * s) // _BLOCK_ROWS,)
    out = pl.pallas_call(
        _copy_body,
        out_shape=jax.ShapeDtypeStruct(x2.shape, x2.dtype),
        grid=grid,
        in_specs=[pl.BlockSpec((_BLOCK_ROWS, d), lambda i: (i, 0))],
        out_specs=pl.BlockSpec((_BLOCK_ROWS, d), lambda i: (i, 0)),
        compiler_params=pltpu.CompilerParams(
            dimension_semantics=("parallel",),
            vmem_limit_bytes=100 * 1024 * 1024,
        ),
    )(x2)
    return out.reshape(b, s, d)
